# baseline (speedup 1.0000x reference)
import functools
import math

import jax
import jax.numpy as jnp
from jax import lax
from jax.experimental import pallas as pl
from jax.experimental.pallas import tpu as pltpu
from jax.experimental.pallas import tpu_sc as plsc

N = 10000
E = 320000
D_IN = 128
D_H = 128
D_OUT = 64

NP = 10240
CHUNK = 128
PHASES = 2
KP = 40
KPS = 48
KA = PHASES * KP
E_PAD = 32 * KA * CHUNK
ROWS_PER_TILE = NP // 16
NBUF = 2

_mesh = plsc.VectorSubcoreMesh(core_axis_name="c", subcore_axis_name="s")


def _fill_rows(ref, nrows, value):
    vv = jnp.full((16,), value, jnp.float32)

    def body(i, _):
        for d in range(8):
            ref[i, pl.ds(16 * d, 16)] = vv
        return 0

    lax.fori_loop(0, nrows, body, 0)


@functools.partial(
    pl.kernel,
    out_type=jax.ShapeDtypeStruct((2, NP, 128), jnp.float32),
    mesh=_mesh,
    scratch_types=[
        pltpu.VMEM((KP, CHUNK), jnp.int32),
        pltpu.VMEM((CHUNK, 128), jnp.float32),
        pltpu.VMEM_SHARED((NP, 128), jnp.float32),
        pltpu.SemaphoreType.DMA,
    ],
)
def _deg_kernel(dst4_hbm, deg_hbm, dst_v, buf_v, acc_s, sem):
    cid = lax.axis_index("c")
    sid = lax.axis_index("s")
    wid = cid * 16 + sid

    _fill_rows(buf_v, CHUNK, 0.0)
    base = sid * ROWS_PER_TILE
    for k in range(ROWS_PER_TILE // CHUNK):
        pltpu.sync_copy(buf_v, acc_s.at[pl.ds(base + k * CHUNK, CHUNK)])
    _fill_rows(buf_v, CHUNK, 1.0)
    plsc.subcore_barrier()

    def deg_group(j0, _):
        for b in range(8):
            pltpu.async_copy(buf_v, acc_s.at[dst_v.at[j0 + b]], sem, add=True)
        for b in range(8):
            pltpu.make_async_copy(buf_v, acc_s.at[dst_v.at[j0 + b]],
                                  sem).wait()
        return 0

    for ph in range(PHASES):
        pltpu.sync_copy(dst4_hbm.at[wid, ph], dst_v)
        lax.fori_loop(0, KP // 8, lambda i, c: deg_group(i * 8, c), 0)
    plsc.subcore_barrier()

    pltpu.sync_copy(
        acc_s.at[pl.ds(base, ROWS_PER_TILE)],
        deg_hbm.at[cid, pl.ds(base, ROWS_PER_TILE)],
    )


@functools.partial(
    pl.kernel,
    out_type=jax.ShapeDtypeStruct((2, NP, 128), jnp.float32),
    mesh=_mesh,
    scratch_types=[
        pltpu.VMEM((KPS, CHUNK), jnp.int32),
        pltpu.VMEM((KP, CHUNK), jnp.int32),
        [pltpu.VMEM((CHUNK, 128), jnp.float32)] * NBUF,
        pltpu.VMEM_SHARED((NP, 128), jnp.float32),
        [pltpu.SemaphoreType.DMA] * NBUF,
        [pltpu.SemaphoreType.DMA] * NBUF,
    ],
)
def _edge_kernel(g_hbm, src4_hbm, dst4_hbm, agg_hbm,
                 src_v, dst_v, rows, acc_s, gsem, ssem):
    cid = lax.axis_index("c")
    sid = lax.axis_index("s")
    wid = cid * 16 + sid

    _fill_rows(rows[0], CHUNK, 0.0)
    base = sid * ROWS_PER_TILE
    for k in range(ROWS_PER_TILE // CHUNK):
        pltpu.sync_copy(rows[0], acc_s.at[pl.ds(base + k * CHUNK, CHUNK)])
    plsc.subcore_barrier()

    def edge_group(j0, _):
        for b in range(NBUF):
            j = j0 + b
            pltpu.make_async_copy(g_hbm.at[src_v.at[j]], rows[b],
                                  gsem[b]).wait()
            pltpu.async_copy(rows[b], acc_s.at[dst_v.at[j]], ssem[b],
                             add=True)
        for b in range(NBUF):
            j = j0 + b
            pltpu.make_async_copy(rows[b], acc_s.at[dst_v.at[j]],
                                  ssem[b]).wait()
            pltpu.async_copy(g_hbm.at[src_v.at[j + NBUF]], rows[b], gsem[b])
        return 0

    for ph in range(PHASES):
        pltpu.sync_copy(src4_hbm.at[wid, ph], src_v)
        pltpu.sync_copy(dst4_hbm.at[wid, ph], dst_v)
        for b in range(NBUF):
            pltpu.async_copy(g_hbm.at[src_v.at[b]], rows[b], gsem[b])
        lax.fori_loop(0, KP // NBUF,
                      lambda i, c: edge_group(i * NBUF, c), 0)
        for b in range(NBUF):
            pltpu.make_async_copy(g_hbm.at[src_v.at[KP + b]], rows[b],
                                  gsem[b]).wait()
    plsc.subcore_barrier()

    pltpu.sync_copy(
        acc_s.at[pl.ds(base, ROWS_PER_TILE)],
        agg_hbm.at[cid, pl.ds(base, ROWS_PER_TILE)],
    )


BLK = 1024
GRID = NP // BLK


def _dinv_blk(d0_ref, d1_ref):
    deg = d0_ref[:, 0] + d1_ref[:, 0] + 1.0
    return 1.0 / jnp.sqrt(deg)


def _tc_first_body(x_ref, w_ref, d0_ref, d1_ref, g_ref):
    dinv = _dinv_blk(d0_ref, d1_ref)
    h = jnp.dot(x_ref[...], w_ref[...], preferred_element_type=jnp.float32)
    g_ref[...] = h * dinv[:, None]


def _tc_mid_body(a0_ref, a1_ref, g_ref, d0_ref, d1_ref,
                 w_ref, s_ref, t_ref, o_ref):
    dinv = _dinv_blk(d0_ref, d1_ref)
    u = (a0_ref[...] + a1_ref[...] + g_ref[...]) * dinv[:, None]
    h = jnp.maximum(u * s_ref[...] + t_ref[...], 0.0)
    o_ref[...] = jnp.dot(
        h, w_ref[...], preferred_element_type=jnp.float32) * dinv[:, None]


def _tc_last_body(a0_ref, a1_ref, g_ref, d0_ref, d1_ref, b_ref, out_ref):
    dinv = _dinv_blk(d0_ref, d1_ref)
    z = ((a0_ref[...] + a1_ref[...] + g_ref[...]) * dinv[:, None])[:, :D_OUT]
    z = z + b_ref[...]
    m = jnp.max(z, axis=1, keepdims=True)
    zs = z - m
    out_ref[...] = zs - jnp.log(jnp.sum(jnp.exp(zs), axis=1, keepdims=True))


def _row_spec(w):
    return pl.BlockSpec((BLK, w), lambda i: (i, 0))


def _full_spec(shape):
    return pl.BlockSpec(shape, lambda i: tuple(0 for _ in shape))


def _mid_call(a2, g, d0, d1, w, s, t):
    return pl.pallas_call(
        _tc_mid_body,
        grid=(GRID,),
        in_specs=[_row_spec(128), _row_spec(128), _row_spec(128),
                  _row_spec(16), _row_spec(16), _full_spec((128, 128)),
                  _full_spec((1, 128)), _full_spec((1, 128))],
        out_specs=_row_spec(128),
        out_shape=jax.ShapeDtypeStruct((NP, 128), jnp.float32),
    )(a2[0], a2[1], g, d0, d1, w, s, t)


def kernel(x, edge_index, W1, b1, g1, be1, W2, b2, g2, be2, W3, b3):
    f32 = jnp.float32
    src = edge_index[0].astype(jnp.int32)
    dst = edge_index[1].astype(jnp.int32)
    pad = jnp.full((E_PAD - E,), N, jnp.int32)
    src4 = jnp.concatenate([src, pad]).reshape(32, PHASES, KP, CHUNK)
    src4 = jnp.pad(src4, ((0, 0), (0, 0), (0, KPS - KP), (0, 0)),
                   constant_values=N)
    dst4 = jnp.concatenate([dst, pad]).reshape(32, PHASES, KP, CHUNK)

    xp = jnp.zeros((NP, D_IN), f32).at[:N].set(x)

    deg2 = _deg_kernel(dst4)
    d0 = deg2[0, :, :16]
    d1 = deg2[1, :, :16]

    bn_c = 1.0 / math.sqrt(1.0 + 1e-5)
    s1 = (g1 * bn_c).reshape(1, D_H)
    t1 = (b1 * g1 * bn_c + be1).reshape(1, D_H)
    s2 = (g2 * bn_c).reshape(1, D_H)
    t2 = (b2 * g2 * bn_c + be2).reshape(1, D_H)
    b3r = b3.reshape(1, D_OUT)
    W3p = jnp.zeros((D_H, 128), f32).at[:, :D_OUT].set(W3)

    ga = pl.pallas_call(
        _tc_first_body,
        grid=(GRID,),
        in_specs=[_row_spec(128), _full_spec((128, 128)),
                  _row_spec(16), _row_spec(16)],
        out_specs=_row_spec(128),
        out_shape=jax.ShapeDtypeStruct((NP, 128), f32),
    )(xp, W1, d0, d1)

    agg1 = _edge_kernel(ga, src4, dst4)
    gb = _mid_call(agg1, ga, d0, d1, W2, s1, t1)
    agg2 = _edge_kernel(gb, src4, dst4)
    gc = _mid_call(agg2, gb, d0, d1, W3p, s2, t2)
    agg3 = _edge_kernel(gc, src4, dst4)

    out = pl.pallas_call(
        _tc_last_body,
        grid=(GRID,),
        in_specs=[_row_spec(128), _row_spec(128), _row_spec(128),
                  _row_spec(16), _row_spec(16), _full_spec((1, 64))],
        out_specs=_row_spec(64),
        out_shape=jax.ShapeDtypeStruct((NP, 64), f32),
    )(agg3[0], agg3[1], gc, d0, d1, b3r)

    return out[:N]


# speedup vs baseline: 2.0873x; 2.0873x over previous
import functools
import math

import jax
import jax.numpy as jnp
from jax import lax
from jax.experimental import pallas as pl
from jax.experimental.pallas import tpu as pltpu
from jax.experimental.pallas import tpu_sc as plsc

N = 10000
E = 320000
D_IN = 128
D_H = 128
D_OUT = 64

NP = 10240
CHUNK = 128
PHASES = 2
KP = 40
KPS = 48
KA = PHASES * KP
E_PAD = 32 * KA * CHUNK
ROWS_PER_TILE = NP // 16
NBUF = 2

_mesh = plsc.VectorSubcoreMesh(core_axis_name="c", subcore_axis_name="s")


def _fill_rows(ref, nrows, value):
    vv = jnp.full((16,), value, jnp.float32)

    def body(i, _):
        for d in range(8):
            ref[i, pl.ds(16 * d, 16)] = vv
        return 0

    lax.fori_loop(0, nrows, body, 0)


@functools.partial(
    pl.kernel,
    out_type=jax.ShapeDtypeStruct((2, NP, 128), jnp.float32),
    mesh=_mesh,
    scratch_types=[
        pltpu.VMEM((KP, CHUNK), jnp.int32),
        pltpu.VMEM((CHUNK, 128), jnp.float32),
        pltpu.VMEM_SHARED((NP, 128), jnp.float32),
        pltpu.SemaphoreType.DMA,
    ],
)
def _deg_kernel(dst4_hbm, deg_hbm, dst_v, buf_v, acc_s, sem):
    cid = lax.axis_index("c")
    sid = lax.axis_index("s")
    wid = cid * 16 + sid

    _fill_rows(buf_v, CHUNK, 0.0)
    base = sid * ROWS_PER_TILE
    for k in range(ROWS_PER_TILE // CHUNK):
        pltpu.sync_copy(buf_v, acc_s.at[pl.ds(base + k * CHUNK, CHUNK)])
    _fill_rows(buf_v, CHUNK, 1.0)
    plsc.subcore_barrier()

    def deg_group(j0, _):
        for b in range(8):
            pltpu.async_copy(buf_v, acc_s.at[dst_v.at[j0 + b]], sem, add=True)
        for b in range(8):
            pltpu.make_async_copy(buf_v, acc_s.at[dst_v.at[j0 + b]],
                                  sem).wait()
        return 0

    for ph in range(PHASES):
        pltpu.sync_copy(dst4_hbm.at[wid, ph], dst_v)
        lax.fori_loop(0, KP // 8, lambda i, c: deg_group(i * 8, c), 0)
    plsc.subcore_barrier()

    pltpu.sync_copy(
        acc_s.at[pl.ds(base, ROWS_PER_TILE)],
        deg_hbm.at[cid, pl.ds(base, ROWS_PER_TILE)],
    )


@functools.partial(
    pl.kernel,
    out_type=jax.ShapeDtypeStruct((2, NP, 128), jnp.float32),
    mesh=_mesh,
    scratch_types=[
        pltpu.VMEM((KPS, CHUNK), jnp.int32),
        pltpu.VMEM((KP, CHUNK), jnp.int32),
        [pltpu.VMEM((CHUNK, 128), jnp.float32)] * NBUF,
        pltpu.VMEM_SHARED((NP, 128), jnp.float32),
        [pltpu.SemaphoreType.DMA] * NBUF,
        [pltpu.SemaphoreType.DMA] * NBUF,
    ],
)
def _edge_kernel(g_hbm, src4_hbm, dst4_hbm, agg_hbm,
                 src_v, dst_v, rows, acc_s, gsem, ssem):
    cid = lax.axis_index("c")
    sid = lax.axis_index("s")
    wid = cid * 16 + sid

    _fill_rows(rows[0], CHUNK, 0.0)
    base = sid * ROWS_PER_TILE
    for k in range(ROWS_PER_TILE // CHUNK):
        pltpu.sync_copy(rows[0], acc_s.at[pl.ds(base + k * CHUNK, CHUNK)])
    plsc.subcore_barrier()

    def edge_pair(j0, _):
        d0 = pltpu.async_copy(g_hbm.at[src_v.at[j0]], rows[0], gsem[0])
        d1 = pltpu.async_copy(g_hbm.at[src_v.at[j0 + 1]], rows[1], gsem[1])
        d0.wait()
        pltpu.sync_copy(rows[0], acc_s.at[dst_v.at[j0]], add=True)
        d1.wait()
        pltpu.sync_copy(rows[1], acc_s.at[dst_v.at[j0 + 1]], add=True)
        return 0

    for ph in range(PHASES):
        pltpu.sync_copy(src4_hbm.at[wid, ph], src_v)
        pltpu.sync_copy(dst4_hbm.at[wid, ph], dst_v)
        lax.fori_loop(0, KP // 2, lambda i, c: edge_pair(i * 2, c), 0)
    plsc.subcore_barrier()

    pltpu.sync_copy(
        acc_s.at[pl.ds(base, ROWS_PER_TILE)],
        agg_hbm.at[cid, pl.ds(base, ROWS_PER_TILE)],
    )


BLK = 1024
GRID = NP // BLK


def _dinv_blk(d0_ref, d1_ref):
    deg = d0_ref[:, 0] + d1_ref[:, 0] + 1.0
    return 1.0 / jnp.sqrt(deg)


def _tc_first_body(x_ref, w_ref, d0_ref, d1_ref, g_ref):
    dinv = _dinv_blk(d0_ref, d1_ref)
    h = jnp.dot(x_ref[...], w_ref[...], preferred_element_type=jnp.float32)
    g_ref[...] = h * dinv[:, None]


def _tc_mid_body(a0_ref, a1_ref, g_ref, d0_ref, d1_ref,
                 w_ref, s_ref, t_ref, o_ref):
    dinv = _dinv_blk(d0_ref, d1_ref)
    u = (a0_ref[...] + a1_ref[...] + g_ref[...]) * dinv[:, None]
    h = jnp.maximum(u * s_ref[...] + t_ref[...], 0.0)
    o_ref[...] = jnp.dot(
        h, w_ref[...], preferred_element_type=jnp.float32) * dinv[:, None]


def _tc_last_body(a0_ref, a1_ref, g_ref, d0_ref, d1_ref, b_ref, out_ref):
    dinv = _dinv_blk(d0_ref, d1_ref)
    z = ((a0_ref[...] + a1_ref[...] + g_ref[...]) * dinv[:, None])[:, :D_OUT]
    z = z + b_ref[...]
    m = jnp.max(z, axis=1, keepdims=True)
    zs = z - m
    out_ref[...] = zs - jnp.log(jnp.sum(jnp.exp(zs), axis=1, keepdims=True))


def _row_spec(w):
    return pl.BlockSpec((BLK, w), lambda i: (i, 0))


def _full_spec(shape):
    return pl.BlockSpec(shape, lambda i: tuple(0 for _ in shape))


def _mid_call(a2, g, d0, d1, w, s, t):
    return pl.pallas_call(
        _tc_mid_body,
        grid=(GRID,),
        in_specs=[_row_spec(128), _row_spec(128), _row_spec(128),
                  _row_spec(16), _row_spec(16), _full_spec((128, 128)),
                  _full_spec((1, 128)), _full_spec((1, 128))],
        out_specs=_row_spec(128),
        out_shape=jax.ShapeDtypeStruct((NP, 128), jnp.float32),
    )(a2[0], a2[1], g, d0, d1, w, s, t)


def kernel(x, edge_index, W1, b1, g1, be1, W2, b2, g2, be2, W3, b3):
    f32 = jnp.float32
    src = edge_index[0].astype(jnp.int32)
    dst = edge_index[1].astype(jnp.int32)
    pad = jnp.full((E_PAD - E,), N, jnp.int32)
    src4 = jnp.concatenate([src, pad]).reshape(32, PHASES, KP, CHUNK)
    src4 = jnp.pad(src4, ((0, 0), (0, 0), (0, KPS - KP), (0, 0)),
                   constant_values=N)
    dst4 = jnp.concatenate([dst, pad]).reshape(32, PHASES, KP, CHUNK)

    xp = jnp.zeros((NP, D_IN), f32).at[:N].set(x)

    deg2 = _deg_kernel(dst4)
    d0 = deg2[0, :, :16]
    d1 = deg2[1, :, :16]

    bn_c = 1.0 / math.sqrt(1.0 + 1e-5)
    s1 = (g1 * bn_c).reshape(1, D_H)
    t1 = (b1 * g1 * bn_c + be1).reshape(1, D_H)
    s2 = (g2 * bn_c).reshape(1, D_H)
    t2 = (b2 * g2 * bn_c + be2).reshape(1, D_H)
    b3r = b3.reshape(1, D_OUT)
    W3p = jnp.zeros((D_H, 128), f32).at[:, :D_OUT].set(W3)

    ga = pl.pallas_call(
        _tc_first_body,
        grid=(GRID,),
        in_specs=[_row_spec(128), _full_spec((128, 128)),
                  _row_spec(16), _row_spec(16)],
        out_specs=_row_spec(128),
        out_shape=jax.ShapeDtypeStruct((NP, 128), f32),
    )(xp, W1, d0, d1)

    agg1 = _edge_kernel(ga, src4, dst4)
    gb = _mid_call(agg1, ga, d0, d1, W2, s1, t1)
    agg2 = _edge_kernel(gb, src4, dst4)
    gc = _mid_call(agg2, gb, d0, d1, W3p, s2, t2)
    agg3 = _edge_kernel(gc, src4, dst4)

    out = pl.pallas_call(
        _tc_last_body,
        grid=(GRID,),
        in_specs=[_row_spec(128), _row_spec(128), _row_spec(128),
                  _row_spec(16), _row_spec(16), _full_spec((1, 64))],
        out_specs=_row_spec(64),
        out_shape=jax.ShapeDtypeStruct((NP, 64), f32),
    )(agg3[0], agg3[1], gc, d0, d1, b3r)

    return out[:N]


# speedup vs baseline: 2.0884x; 1.0005x over previous
import functools
import math

import jax
import jax.numpy as jnp
from jax import lax
from jax.experimental import pallas as pl
from jax.experimental.pallas import tpu as pltpu
from jax.experimental.pallas import tpu_sc as plsc

N = 10000
E = 320000
D_IN = 128
D_H = 128
D_OUT = 64

NP = 10240
CHUNK = 128
PHASES = 2
KP = 40
KPS = 48
KA = PHASES * KP
E_PAD = 32 * KA * CHUNK
ROWS_PER_TILE = NP // 16
NBUF = 2

_mesh = plsc.VectorSubcoreMesh(core_axis_name="c", subcore_axis_name="s")


def _fill_rows(ref, nrows, value):
    vv = jnp.full((16,), value, jnp.float32)

    def body(i, _):
        for d in range(8):
            ref[i, pl.ds(16 * d, 16)] = vv
        return 0

    lax.fori_loop(0, nrows, body, 0)


@functools.partial(
    pl.kernel,
    out_type=jax.ShapeDtypeStruct((2, NP, 128), jnp.float32),
    mesh=_mesh,
    scratch_types=[
        pltpu.VMEM((KP, CHUNK), jnp.int32),
        pltpu.VMEM((CHUNK, 128), jnp.float32),
        pltpu.VMEM_SHARED((NP, 128), jnp.float32),
        pltpu.SemaphoreType.DMA,
    ],
)
def _deg_kernel(dst4_hbm, deg_hbm, dst_v, buf_v, acc_s, sem):
    cid = lax.axis_index("c")
    sid = lax.axis_index("s")
    wid = cid * 16 + sid

    _fill_rows(buf_v, CHUNK, 0.0)
    base = sid * ROWS_PER_TILE
    for k in range(ROWS_PER_TILE // CHUNK):
        pltpu.sync_copy(buf_v, acc_s.at[pl.ds(base + k * CHUNK, CHUNK)])
    _fill_rows(buf_v, CHUNK, 1.0)
    plsc.subcore_barrier()

    def deg_group(j0, _):
        for b in range(8):
            pltpu.async_copy(buf_v, acc_s.at[dst_v.at[j0 + b]], sem, add=True)
        for b in range(8):
            pltpu.make_async_copy(buf_v, acc_s.at[dst_v.at[j0 + b]],
                                  sem).wait()
        return 0

    for ph in range(PHASES):
        pltpu.sync_copy(dst4_hbm.at[wid, ph], dst_v)
        lax.fori_loop(0, KP // 8, lambda i, c: deg_group(i * 8, c), 0)
    plsc.subcore_barrier()

    pltpu.sync_copy(
        acc_s.at[pl.ds(base, ROWS_PER_TILE)],
        deg_hbm.at[cid, pl.ds(base, ROWS_PER_TILE)],
    )


@functools.partial(
    pl.kernel,
    out_type=jax.ShapeDtypeStruct((2, NP, 128), jnp.float32),
    mesh=_mesh,
    scratch_types=[
        pltpu.VMEM((KPS, CHUNK), jnp.int32),
        pltpu.VMEM((KP, CHUNK), jnp.int32),
        [pltpu.VMEM((CHUNK, 128), jnp.float32)] * NBUF,
        pltpu.VMEM_SHARED((NP, 128), jnp.float32),
        [pltpu.SemaphoreType.DMA] * NBUF,
        [pltpu.SemaphoreType.DMA] * NBUF,
    ],
)
def _edge_kernel(g_hbm, src4_hbm, dst4_hbm, agg_hbm,
                 src_v, dst_v, rows, acc_s, gsem, ssem):
    cid = lax.axis_index("c")
    sid = lax.axis_index("s")
    wid = cid * 16 + sid

    _fill_rows(rows[0], CHUNK, 0.0)
    base = sid * ROWS_PER_TILE
    for k in range(ROWS_PER_TILE // CHUNK):
        pltpu.sync_copy(rows[0], acc_s.at[pl.ds(base + k * CHUNK, CHUNK)])
    plsc.subcore_barrier()

    def edge_pair(j0, _):
        H = CHUNK // 2
        ds = []
        for b in range(NBUF):
            j = j0 + b
            ds.append(pltpu.async_copy(
                g_hbm.at[src_v.at[j, pl.ds(0, H)]],
                rows[b].at[pl.ds(0, H)], gsem[b]))
            ds.append(pltpu.async_copy(
                g_hbm.at[src_v.at[j, pl.ds(H, H)]],
                rows[b].at[pl.ds(H, H)], gsem[b]))
        for b in range(NBUF):
            ds[2 * b].wait()
            ds[2 * b + 1].wait()
            pltpu.sync_copy(rows[b], acc_s.at[dst_v.at[j0 + b]], add=True)
        return 0

    for ph in range(PHASES):
        pltpu.sync_copy(src4_hbm.at[wid, ph], src_v)
        pltpu.sync_copy(dst4_hbm.at[wid, ph], dst_v)
        lax.fori_loop(0, KP // 2, lambda i, c: edge_pair(i * 2, c), 0)
    plsc.subcore_barrier()

    pltpu.sync_copy(
        acc_s.at[pl.ds(base, ROWS_PER_TILE)],
        agg_hbm.at[cid, pl.ds(base, ROWS_PER_TILE)],
    )


BLK = 1024
GRID = NP // BLK


def _dinv_blk(d0_ref, d1_ref):
    deg = d0_ref[:, 0] + d1_ref[:, 0] + 1.0
    return 1.0 / jnp.sqrt(deg)


def _tc_first_body(x_ref, w_ref, d0_ref, d1_ref, g_ref):
    dinv = _dinv_blk(d0_ref, d1_ref)
    h = jnp.dot(x_ref[...], w_ref[...], preferred_element_type=jnp.float32)
    g_ref[...] = h * dinv[:, None]


def _tc_mid_body(a0_ref, a1_ref, g_ref, d0_ref, d1_ref,
                 w_ref, s_ref, t_ref, o_ref):
    dinv = _dinv_blk(d0_ref, d1_ref)
    u = (a0_ref[...] + a1_ref[...] + g_ref[...]) * dinv[:, None]
    h = jnp.maximum(u * s_ref[...] + t_ref[...], 0.0)
    o_ref[...] = jnp.dot(
        h, w_ref[...], preferred_element_type=jnp.float32) * dinv[:, None]


def _tc_last_body(a0_ref, a1_ref, g_ref, d0_ref, d1_ref, b_ref, out_ref):
    dinv = _dinv_blk(d0_ref, d1_ref)
    z = ((a0_ref[...] + a1_ref[...] + g_ref[...]) * dinv[:, None])[:, :D_OUT]
    z = z + b_ref[...]
    m = jnp.max(z, axis=1, keepdims=True)
    zs = z - m
    out_ref[...] = zs - jnp.log(jnp.sum(jnp.exp(zs), axis=1, keepdims=True))


def _row_spec(w):
    return pl.BlockSpec((BLK, w), lambda i: (i, 0))


def _full_spec(shape):
    return pl.BlockSpec(shape, lambda i: tuple(0 for _ in shape))


def _mid_call(a2, g, d0, d1, w, s, t):
    return pl.pallas_call(
        _tc_mid_body,
        grid=(GRID,),
        in_specs=[_row_spec(128), _row_spec(128), _row_spec(128),
                  _row_spec(16), _row_spec(16), _full_spec((128, 128)),
                  _full_spec((1, 128)), _full_spec((1, 128))],
        out_specs=_row_spec(128),
        out_shape=jax.ShapeDtypeStruct((NP, 128), jnp.float32),
    )(a2[0], a2[1], g, d0, d1, w, s, t)


def kernel(x, edge_index, W1, b1, g1, be1, W2, b2, g2, be2, W3, b3):
    f32 = jnp.float32
    src = edge_index[0].astype(jnp.int32)
    dst = edge_index[1].astype(jnp.int32)
    pad = jnp.full((E_PAD - E,), N, jnp.int32)
    src4 = jnp.concatenate([src, pad]).reshape(32, PHASES, KP, CHUNK)
    src4 = jnp.pad(src4, ((0, 0), (0, 0), (0, KPS - KP), (0, 0)),
                   constant_values=N)
    dst4 = jnp.concatenate([dst, pad]).reshape(32, PHASES, KP, CHUNK)

    xp = jnp.zeros((NP, D_IN), f32).at[:N].set(x)

    deg2 = _deg_kernel(dst4)
    d0 = deg2[0, :, :16]
    d1 = deg2[1, :, :16]

    bn_c = 1.0 / math.sqrt(1.0 + 1e-5)
    s1 = (g1 * bn_c).reshape(1, D_H)
    t1 = (b1 * g1 * bn_c + be1).reshape(1, D_H)
    s2 = (g2 * bn_c).reshape(1, D_H)
    t2 = (b2 * g2 * bn_c + be2).reshape(1, D_H)
    b3r = b3.reshape(1, D_OUT)
    W3p = jnp.zeros((D_H, 128), f32).at[:, :D_OUT].set(W3)

    ga = pl.pallas_call(
        _tc_first_body,
        grid=(GRID,),
        in_specs=[_row_spec(128), _full_spec((128, 128)),
                  _row_spec(16), _row_spec(16)],
        out_specs=_row_spec(128),
        out_shape=jax.ShapeDtypeStruct((NP, 128), f32),
    )(xp, W1, d0, d1)

    agg1 = _edge_kernel(ga, src4, dst4)
    gb = _mid_call(agg1, ga, d0, d1, W2, s1, t1)
    agg2 = _edge_kernel(gb, src4, dst4)
    gc = _mid_call(agg2, gb, d0, d1, W3p, s2, t2)
    agg3 = _edge_kernel(gc, src4, dst4)

    out = pl.pallas_call(
        _tc_last_body,
        grid=(GRID,),
        in_specs=[_row_spec(128), _row_spec(128), _row_spec(128),
                  _row_spec(16), _row_spec(16), _full_spec((1, 64))],
        out_specs=_row_spec(64),
        out_shape=jax.ShapeDtypeStruct((NP, 64), f32),
    )(agg3[0], agg3[1], gc, d0, d1, b3r)

    return out[:N]
